# baseline (speedup 1.0000x reference)
import functools

import jax
import jax.numpy as jnp
from jax import lax
from jax.experimental import pallas as pl
from jax.experimental.pallas import tpu as pltpu
from jax.experimental.pallas import tpu_sc as plsc

HEADS = 4
HD = 16
F32 = jnp.float32
I32 = jnp.int32

NC = 2
NS = 16
NW = NC * NS

_SC_PARAMS = pltpu.CompilerParams(use_tc_tiling_on_sc=False)


def _prep_layer(x_in, w_split, acat_split, npad, from_split):
    B = 512
    nb = pl.cdiv(npad, B)

    def body(x_ref, w_ref, a_ref, xw_ref, sd_ref):
        h = pl.program_id(1)
        if from_split:
            xb = x_ref[...]
            xb = jnp.concatenate([xb[0], xb[1]], axis=-1)
            xb = jnp.where(xb > 0, xb, jnp.exp(xb) - 1.0)
        else:
            xb = x_ref[...]
        xw = jnp.dot(xb, w_ref[0], preferred_element_type=F32)
        xw_ref[...] = xw[None]
        part = jnp.dot(xw, a_ref[0], preferred_element_type=F32)

        @pl.when(h == 0)
        def _():
            sd_ref[...] = part

        @pl.when(h == 1)
        def _():
            sd_ref[...] = sd_ref[...] + part

    if from_split:
        x_spec = pl.BlockSpec((2, B, 32), lambda i, h: (0, i, 0))
    else:
        x_spec = pl.BlockSpec((B, 64), lambda i, h: (i, 0))
    return pl.pallas_call(
        body,
        grid=(nb, 2),
        in_specs=[
            x_spec,
            pl.BlockSpec((1, 64, 32), lambda i, h: (h, 0, 0)),
            pl.BlockSpec((1, 32, 16), lambda i, h: (h, 0, 0)),
        ],
        out_specs=[
            pl.BlockSpec((1, B, 32), lambda i, h: (h, i, 0)),
            pl.BlockSpec((B, 16), lambda i, h: (i, 0)),
        ],
        out_shape=[
            jax.ShapeDtypeStruct((2, npad, 32), F32),
            jax.ShapeDtypeStruct((npad, 16), F32),
        ],
    )(x_in, w_split, acat_split)


def _edge_logits(src, dst, sdss_flat, e, npad):
    ch = 128
    n_total = e // ch
    seg = npad
    mesh = plsc.VectorSubcoreMesh(core_axis_name="c", subcore_axis_name="s")

    @functools.partial(
        pl.kernel,
        out_type=jax.ShapeDtypeStruct((HEADS * e,), F32),
        mesh=mesh,
        compiler_params=_SC_PARAMS,
        scratch_types=[
            pltpu.VMEM_SHARED((npad * 16,), F32),
            pltpu.VMEM((ch,), I32),
            pltpu.VMEM((ch,), I32),
        ] + [pltpu.VMEM((ch,), I32) for _ in range(8)]
          + [pltpu.VMEM((ch,), F32) for _ in range(8)]
          + [
            pltpu.VMEM((HEADS, ch), F32),
            pltpu.SemaphoreType.DMA,
            pltpu.SemaphoreType.DMA,
        ],
    )
    def k(src_h, dst_h, sdss_h, lt_h, sd_sh, sidx, didx,
          db0, db1, db2, db3, sb0, sb1, sb2, sb3,
          dg0, dg1, dg2, dg3, sg0, sg1, sg2, sg3,
          ltv, sem1, sem2):
        dbs = (db0, db1, db2, db3)
        sbs = (sb0, sb1, sb2, sb3)
        dgs = (dg0, dg1, dg2, dg3)
        sgs = (sg0, sg1, sg2, sg3)
        c = lax.axis_index("c")
        s = lax.axis_index("s")
        wid = s * NC + c
        pltpu.sync_copy(sdss_h.at[pl.ds(s * seg, seg)],
                        sd_sh.at[pl.ds(s * seg, seg)])
        plsc.subcore_barrier()
        n_my = (n_total - wid + NW - 1) // NW

        def chunk(i, _):
            ci = wid + i * NW
            e0 = pl.multiple_of(ci * ch, 128)
            cS = pltpu.async_copy(src_h.at[pl.ds(e0, ch)], sidx, sem1)
            cD = pltpu.async_copy(dst_h.at[pl.ds(e0, ch)], didx, sem2)
            cS.wait()
            cD.wait()

            def mkidx(j, _):
                o = j * 16
                dv = didx[pl.ds(o, 16)] * 16
                sv = sidx[pl.ds(o, 16)] * 16
                for h in range(HEADS):
                    dbs[h][pl.ds(o, 16)] = dv + h
                    sbs[h][pl.ds(o, 16)] = sv + (4 + h)
                return 0

            lax.fori_loop(0, ch // 16, mkidx, 0)
            cps = ([pltpu.async_copy(sd_sh.at[dbs[h]], dgs[h], sem1)
                    for h in range(HEADS)]
                   + [pltpu.async_copy(sd_sh.at[sbs[h]], sgs[h], sem2)
                      for h in range(HEADS)])
            for cp in cps:
                cp.wait()

            def lrelu(j, _):
                o = j * 16
                for h in range(HEADS):
                    v = dgs[h][pl.ds(o, 16)] + sgs[h][pl.ds(o, 16)]
                    ltv[h, pl.ds(o, 16)] = jnp.maximum(v, 0.2 * v)
                return 0

            lax.fori_loop(0, ch // 16, lrelu, 0)
            cO = [pltpu.async_copy(
                ltv.at[h],
                lt_h.at[pl.ds(pl.multiple_of(h * e + e0, 128), ch)], sem1)
                for h in range(HEADS)]
            for cp in cO:
                cp.wait()
            return 0

        lax.fori_loop(0, n_my, chunk, 0)

    return k(src, dst, sdss_flat)


def _softmax_stats(lt3, e):
    rows = e // 128

    def body(lt_ref, ms_ref):
        blk = lt_ref[...]
        m_lane = jnp.max(blk, axis=1)
        m_f = jnp.max(m_lane, axis=1, keepdims=True)
        s_lane = jnp.sum(jnp.exp(blk - m_f[:, :, None]), axis=1)
        s_f = jnp.sum(s_lane, axis=1, keepdims=True)
        ms_ref[...] = jnp.concatenate(
            [jnp.broadcast_to(m_f, (4, 128)),
             jnp.broadcast_to(s_f, (4, 128))], axis=0)

    return pl.pallas_call(
        body,
        in_specs=[pl.BlockSpec((4, rows, 128), lambda: (0, 0, 0))],
        out_specs=pl.BlockSpec((8, 128), lambda: (0, 0)),
        out_shape=jax.ShapeDtypeStruct((8, 128), F32),
    )(lt3)


def _lane_max(v):
    r = v[0]
    for i in range(1, 16):
        r = jnp.maximum(r, v[i])
    return r


def _lane_sum(v):
    r = v[0]
    for i in range(1, 16):
        r = r + v[i]
    return r


def _message_pass(xw2flat, src, dst, lt, e, npad):
    ch = 128
    nb4 = 4
    n_total = e // ch
    rows_sub = npad // NS
    zr = 92
    nz = rows_sub // zr
    mesh = plsc.VectorSubcoreMesh(core_axis_name="c", subcore_axis_name="s")

    @functools.partial(
        pl.kernel,
        out_type=jax.ShapeDtypeStruct((2, npad, 32), F32),
        mesh=mesh,
        compiler_params=_SC_PARAMS,
        scratch_types=(
            [pltpu.VMEM_SHARED((npad, 32), F32)]
            + [pltpu.VMEM((ch,), I32) for _ in range(2 * nb4)]
            + [pltpu.VMEM((ch, 32), F32) for _ in range(nb4)]
            + [pltpu.VMEM((2 * ch,), F32) for _ in range(nb4)]
            + [pltpu.VMEM_SHARED((80,), F32)]
            + [pltpu.VMEM((2000,), F32)]
            + [pltpu.VMEM((80,), F32)]
            + [pltpu.VMEM((16,), I32)]
            + [pltpu.SemaphoreType.DMA for _ in range(4)]
        ),
    )
    def k(xw_h, src_h, dst_h, lt_h, out_h, acc_sh,
          si0, si1, si2, si3, di0, di1, di2, di3,
          ro0, ro1, ro2, ro3, av0, av1, av2, av3,
          st_sh, stbuf, stv, stidx, sem_idx, sem_av, sem_gat, sem_sca):
        sis = (si0, si1, si2, si3)
        dis = (di0, di1, di2, di3)
        ros = (ro0, ro1, ro2, ro3)
        avs = (av0, av1, av2, av3)
        c = lax.axis_index("c")
        s = lax.axis_index("s")
        zv = jnp.zeros((16,), F32)

        def zrow(i, _):
            ro0[i, pl.ds(0, 16)] = zv
            ro0[i, pl.ds(16, 16)] = zv
            return 0

        lax.fori_loop(0, zr, zrow, 0)

        def zchunk(i, _):
            pltpu.sync_copy(ro0.at[pl.ds(0, zr)],
                            acc_sh.at[pl.ds(s * rows_sub + i * zr, zr)])
            return 0

        lax.fori_loop(0, nz, zchunk, 0)
        st_ch = 2000
        n_st = (e // NS) // st_ch
        ninf = jnp.float32(-3.0e38)
        tile_ms = []
        for hl in range(2):
            base_q = pl.multiple_of((2 * c + hl) * e + s * (e // NS), 8)

            def maxc(i, m_v):
                pltpu.sync_copy(lt_h.at[pl.ds(base_q + i * st_ch, st_ch)],
                                stbuf)

                def mv(j, m_v):
                    return jnp.maximum(m_v, stbuf[pl.ds(j * 16, 16)])

                return lax.fori_loop(0, st_ch // 16, mv, m_v)

            m_v = lax.fori_loop(0, n_st, maxc, jnp.full((16,), ninf, F32))
            m_t = _lane_max(m_v)

            def sumc(i, s_v):
                pltpu.sync_copy(lt_h.at[pl.ds(base_q + i * st_ch, st_ch)],
                                stbuf)

                def sv(j, s_v):
                    return s_v + jnp.exp(stbuf[pl.ds(j * 16, 16)] - m_t)

                return lax.fori_loop(0, st_ch // 16, sv, s_v)

            s_v = lax.fori_loop(0, n_st, sumc, jnp.zeros((16,), F32))
            s_t = _lane_sum(s_v)
            tile_ms.append((m_t, s_t))
        lanes = lax.iota(I32, 16)
        stidx[pl.ds(0, 16)] = jnp.where(lanes < 4, lanes * 16 + s, 64 + lanes)
        vals = jnp.zeros((16,), F32)
        vals = jnp.where(lanes == 0, tile_ms[0][0], vals)
        vals = jnp.where(lanes == 1, tile_ms[0][1], vals)
        vals = jnp.where(lanes == 2, tile_ms[1][0], vals)
        vals = jnp.where(lanes == 3, tile_ms[1][1], vals)
        stv[pl.ds(0, 16)] = vals
        plsc.subcore_barrier()
        pltpu.sync_copy(stv.at[pl.ds(0, 16)], st_sh.at[stidx])
        plsc.subcore_barrier()
        pltpu.sync_copy(st_sh, stv)
        m_all0 = stv[pl.ds(0, 16)]
        s_all0 = stv[pl.ds(16, 16)]
        m_all1 = stv[pl.ds(32, 16)]
        s_all1 = stv[pl.ds(48, 16)]
        m0 = _lane_max(m_all0)
        d0 = _lane_sum(s_all0 * jnp.exp(m_all0 - m0))
        m1 = _lane_max(m_all1)
        d1 = _lane_sum(s_all1 * jnp.exp(m_all1 - m1))
        plsc.subcore_barrier()
        rowbase = c * npad
        n_my = (n_total - s + NS - 1) // NS
        n_grp = (n_total + NS - 1) // NS
        n_grp = (n_grp + nb4 - 1) // nb4

        def group(g, _):
            ks = [g * nb4 + b for b in range(nb4)]
            safe = s * ch
            e0s = [pl.multiple_of(
                jnp.where(kk < n_my, (s + kk * NS) * ch, safe), 128)
                for kk in ks]
            cS = [pltpu.async_copy(src_h.at[pl.ds(e0s[b], ch)], sis[b],
                                   sem_idx) for b in range(nb4)]
            cD = [pltpu.async_copy(dst_h.at[pl.ds(e0s[b], ch)], dis[b],
                                   sem_idx) for b in range(nb4)]
            cA = [[pltpu.async_copy(
                lt_h.at[pl.ds(pl.multiple_of(
                    (2 * c + hl) * e + e0s[b], 128), ch)],
                avs[b].at[pl.ds(hl * ch, ch)], sem_av)
                for hl in range(2)] for b in range(nb4)]
            cG = [None] * nb4
            for b in range(nb4):
                cS[b].wait()

                def addoff(j, _):
                    o = j * 16
                    sis[b][pl.ds(o, 16)] = sis[b][pl.ds(o, 16)] + rowbase
                    return 0

                lax.fori_loop(0, ch // 16, addoff, 0)
                cG[b] = pltpu.async_copy(xw_h.at[sis[b]], ros[b], sem_gat)
            for b in range(nb4):
                cA[b][0].wait()
                cA[b][1].wait()
                for hl, (m_q, d_q) in enumerate(((m0, d0), (m1, d1))):

                    def ablk(j, _):
                        o = hl * ch + j * 16
                        v = avs[b][pl.ds(o, 16)]
                        avs[b][pl.ds(o, 16)] = jnp.exp(v - m_q) / d_q
                        return 0

                    lax.fori_loop(0, ch // 16, ablk, 0)

                @pl.when(ks[b] >= n_my)
                def _(b=b):
                    zv16 = jnp.zeros((16,), F32)

                    def zab(j, _):
                        avs[b][pl.ds(j * 16, 16)] = zv16
                        return 0

                    lax.fori_loop(0, (2 * ch) // 16, zab, 0)
            cW = [None] * nb4
            for b in range(nb4):
                cG[b].wait()

                def scale(j, _):
                    o = j * 16
                    a16_0 = avs[b][pl.ds(o, 16)]
                    a16_1 = avs[b][pl.ds(ch + o, 16)]
                    for t in range(16):
                        r = o + t
                        ros[b][r, pl.ds(0, 16)] = (
                            ros[b][r, pl.ds(0, 16)] * a16_0[t])
                        ros[b][r, pl.ds(16, 16)] = (
                            ros[b][r, pl.ds(16, 16)] * a16_1[t])
                    return 0

                lax.fori_loop(0, ch // 16, scale, 0)
                cD[b].wait()
                cW[b] = pltpu.async_copy(ros[b], acc_sh.at[dis[b]],
                                         sem_sca, add=True)
            for b in range(nb4):
                cW[b].wait()
            return 0

        lax.fori_loop(0, n_grp, group, 0)
        plsc.subcore_barrier()
        pltpu.sync_copy(acc_sh.at[pl.ds(s * rows_sub, rows_sub)],
                        out_h.at[c, pl.ds(s * rows_sub, rows_sub)])

    return k(xw2flat, src, dst, lt)


def _final_mean(x0, os1, os2, n):
    B = 512
    nb = pl.cdiv(n, B)

    def body(x0_ref, o1_ref, o2_ref, out_ref):
        def cat_elu(r):
            v = jnp.concatenate([r[0], r[1]], axis=-1)
            return jnp.where(v > 0, v, jnp.exp(v) - 1.0)

        out_ref[...] = (x0_ref[...] + cat_elu(o1_ref[...])
                        + cat_elu(o2_ref[...])) * (1.0 / 3.0)

    return pl.pallas_call(
        body,
        grid=(nb,),
        in_specs=[
            pl.BlockSpec((B, 64), lambda i: (i, 0)),
            pl.BlockSpec((2, B, 32), lambda i: (0, i, 0)),
            pl.BlockSpec((2, B, 32), lambda i: (0, i, 0)),
        ],
        out_specs=pl.BlockSpec((B, 64), lambda i: (i, 0)),
        out_shape=jax.ShapeDtypeStruct((n, 64), F32),
    )(x0, os1, os2)


def _acat_split(a):
    ad = a[0, :, :HD]
    asr = a[0, :, HD:]
    eye = jnp.eye(HEADS, dtype=F32)
    blk_d = (eye[:, None, :] * ad[:, :, None]).reshape(64, HEADS)
    blk_s = (eye[:, None, :] * asr[:, :, None]).reshape(64, HEADS)
    acat = jnp.concatenate(
        [blk_d, blk_s, jnp.zeros((64, 8), F32)], axis=1)
    return acat.reshape(2, 32, 16)


def kernel(edge_index, user_table, item_table, cat_table, W1, a1, W2, a2):
    src = edge_index[0]
    dst = edge_index[1]
    e = src.shape[0]
    n = user_table.shape[0] + item_table.shape[0] + cat_table.shape[0]
    npad = ((n + 127) // 128) * 128
    n_users = user_table.shape[0]
    n_items = item_table.shape[0]

    x0 = jnp.concatenate([user_table, item_table, cat_table], axis=0)

    def layer(x_in, W, a, from_split):
        w_split = jnp.stack([W[:, 0:32], W[:, 32:64]])
        xw2, sdss = _prep_layer(x_in, w_split, _acat_split(a), npad,
                                from_split)
        lt = _edge_logits(src, dst, sdss.reshape(npad * 16), e, npad)
        return _message_pass(xw2.reshape(2 * npad, 32), src, dst, lt,
                             e, npad)

    os1 = layer(x0, W1, a1, False)
    os2 = layer(os1, W2, a2, True)
    final = _final_mean(x0, os1, os2, n)
    return (final[:n_users], final[n_users:n_users + n_items])


# speedup vs baseline: 1.1227x; 1.1227x over previous
import functools

import jax
import jax.numpy as jnp
from jax import lax
from jax.experimental import pallas as pl
from jax.experimental.pallas import tpu as pltpu
from jax.experimental.pallas import tpu_sc as plsc

HEADS = 4
HD = 16
F32 = jnp.float32
I32 = jnp.int32

NC = 2
NS = 16
NW = NC * NS

_SC_PARAMS = pltpu.CompilerParams(use_tc_tiling_on_sc=False)


def _prep_layer(x_in, w_split, acat_split, npad, from_split):
    B = 512
    nb = pl.cdiv(npad, B)

    def body(x_ref, w_ref, a_ref, xw_ref, sd_ref):
        h = pl.program_id(1)
        if from_split:
            xb = x_ref[...]
            xb = jnp.concatenate([xb[0], xb[1]], axis=-1)
            xb = jnp.where(xb > 0, xb, jnp.exp(xb) - 1.0)
        else:
            xb = x_ref[...]
        xw = jnp.dot(xb, w_ref[0], preferred_element_type=F32)
        xw_ref[...] = xw[None]
        part = jnp.dot(xw, a_ref[0], preferred_element_type=F32)

        @pl.when(h == 0)
        def _():
            sd_ref[...] = part

        @pl.when(h == 1)
        def _():
            sd_ref[...] = sd_ref[...] + part

    if from_split:
        x_spec = pl.BlockSpec((2, B, 32), lambda i, h: (0, i, 0))
    else:
        x_spec = pl.BlockSpec((B, 64), lambda i, h: (i, 0))
    return pl.pallas_call(
        body,
        grid=(nb, 2),
        in_specs=[
            x_spec,
            pl.BlockSpec((1, 64, 32), lambda i, h: (h, 0, 0)),
            pl.BlockSpec((1, 32, 16), lambda i, h: (h, 0, 0)),
        ],
        out_specs=[
            pl.BlockSpec((1, B, 32), lambda i, h: (h, i, 0)),
            pl.BlockSpec((B, 16), lambda i, h: (i, 0)),
        ],
        out_shape=[
            jax.ShapeDtypeStruct((2, npad, 32), F32),
            jax.ShapeDtypeStruct((npad, 16), F32),
        ],
    )(x_in, w_split, acat_split)


def _edge_logits(src, dst, sdss_flat, e, npad):
    ch = 128
    n_total = e // ch
    seg = npad
    mesh = plsc.VectorSubcoreMesh(core_axis_name="c", subcore_axis_name="s")

    @functools.partial(
        pl.kernel,
        out_type=jax.ShapeDtypeStruct((HEADS * e,), F32),
        mesh=mesh,
        compiler_params=_SC_PARAMS,
        scratch_types=[
            pltpu.VMEM_SHARED((npad * 16,), F32),
            pltpu.VMEM((ch,), I32),
            pltpu.VMEM((ch,), I32),
        ] + [pltpu.VMEM((ch,), I32) for _ in range(8)]
          + [pltpu.VMEM((ch,), F32) for _ in range(8)]
          + [
            pltpu.VMEM((HEADS, ch), F32),
            pltpu.SemaphoreType.DMA,
            pltpu.SemaphoreType.DMA,
        ],
    )
    def k(src_h, dst_h, sdss_h, lt_h, sd_sh, sidx, didx,
          db0, db1, db2, db3, sb0, sb1, sb2, sb3,
          dg0, dg1, dg2, dg3, sg0, sg1, sg2, sg3,
          ltv, sem1, sem2):
        dbs = (db0, db1, db2, db3)
        sbs = (sb0, sb1, sb2, sb3)
        dgs = (dg0, dg1, dg2, dg3)
        sgs = (sg0, sg1, sg2, sg3)
        c = lax.axis_index("c")
        s = lax.axis_index("s")
        wid = s * NC + c
        pltpu.sync_copy(sdss_h.at[pl.ds(s * seg, seg)],
                        sd_sh.at[pl.ds(s * seg, seg)])
        plsc.subcore_barrier()
        n_my = (n_total - wid + NW - 1) // NW

        def chunk(i, _):
            ci = wid + i * NW
            e0 = pl.multiple_of(ci * ch, 128)
            cS = pltpu.async_copy(src_h.at[pl.ds(e0, ch)], sidx, sem1)
            cD = pltpu.async_copy(dst_h.at[pl.ds(e0, ch)], didx, sem2)
            cS.wait()
            cD.wait()

            def mkidx(j, _):
                o = j * 16
                dv = didx[pl.ds(o, 16)] * 16
                sv = sidx[pl.ds(o, 16)] * 16
                for h in range(HEADS):
                    dbs[h][pl.ds(o, 16)] = dv + h
                    sbs[h][pl.ds(o, 16)] = sv + (4 + h)
                return 0

            lax.fori_loop(0, ch // 16, mkidx, 0)
            cps = ([pltpu.async_copy(sd_sh.at[dbs[h]], dgs[h], sem1)
                    for h in range(HEADS)]
                   + [pltpu.async_copy(sd_sh.at[sbs[h]], sgs[h], sem2)
                      for h in range(HEADS)])
            for cp in cps:
                cp.wait()

            def lrelu(j, _):
                o = j * 16
                for h in range(HEADS):
                    v = dgs[h][pl.ds(o, 16)] + sgs[h][pl.ds(o, 16)]
                    ltv[h, pl.ds(o, 16)] = jnp.maximum(v, 0.2 * v)
                return 0

            lax.fori_loop(0, ch // 16, lrelu, 0)
            cO = [pltpu.async_copy(
                ltv.at[h],
                lt_h.at[pl.ds(pl.multiple_of(h * e + e0, 128), ch)], sem1)
                for h in range(HEADS)]
            for cp in cO:
                cp.wait()
            return 0

        lax.fori_loop(0, n_my, chunk, 0)

    return k(src, dst, sdss_flat)


def _softmax_stats(lt3, e):
    rows = e // 128

    def body(lt_ref, ms_ref):
        blk = lt_ref[...]
        m_lane = jnp.max(blk, axis=1)
        m_f = jnp.max(m_lane, axis=1, keepdims=True)
        s_lane = jnp.sum(jnp.exp(blk - m_f[:, :, None]), axis=1)
        s_f = jnp.sum(s_lane, axis=1, keepdims=True)
        ms_ref[...] = jnp.concatenate(
            [jnp.broadcast_to(m_f, (4, 128)),
             jnp.broadcast_to(s_f, (4, 128))], axis=0)

    return pl.pallas_call(
        body,
        in_specs=[pl.BlockSpec((4, rows, 128), lambda: (0, 0, 0))],
        out_specs=pl.BlockSpec((8, 128), lambda: (0, 0)),
        out_shape=jax.ShapeDtypeStruct((8, 128), F32),
    )(lt3)


def _message_pass(xw2flat, src, dst, lt, ms32, e, npad):
    ch = 128
    nb4 = 4
    n_total = e // ch
    rows_sub = npad // NS
    zr = 92
    nz = rows_sub // zr
    mesh = plsc.VectorSubcoreMesh(core_axis_name="c", subcore_axis_name="s")

    @functools.partial(
        pl.kernel,
        out_type=jax.ShapeDtypeStruct((2, npad, 32), F32),
        mesh=mesh,
        compiler_params=_SC_PARAMS,
        scratch_types=(
            [pltpu.VMEM_SHARED((npad, 32), F32)]
            + [pltpu.VMEM((ch,), I32) for _ in range(2 * nb4)]
            + [pltpu.VMEM((ch, 32), F32) for _ in range(nb4)]
            + [pltpu.VMEM((2 * ch,), F32) for _ in range(nb4)]
            + [pltpu.VMEM((32,), F32)]
            + [pltpu.SemaphoreType.DMA for _ in range(4)]
        ),
    )
    def k(xw_h, src_h, dst_h, lt_h, ms_h, out_h, acc_sh,
          si0, si1, si2, si3, di0, di1, di2, di3,
          ro0, ro1, ro2, ro3, av0, av1, av2, av3,
          msv, sem_idx, sem_av, sem_gat, sem_sca):
        sis = (si0, si1, si2, si3)
        dis = (di0, di1, di2, di3)
        ros = (ro0, ro1, ro2, ro3)
        avs = (av0, av1, av2, av3)
        c = lax.axis_index("c")
        s = lax.axis_index("s")
        zv = jnp.zeros((16,), F32)

        def zrow(i, _):
            ro0[i, pl.ds(0, 16)] = zv
            ro0[i, pl.ds(16, 16)] = zv
            return 0

        lax.fori_loop(0, zr, zrow, 0)

        def zchunk(i, _):
            pltpu.sync_copy(ro0.at[pl.ds(0, zr)],
                            acc_sh.at[pl.ds(s * rows_sub + i * zr, zr)])
            return 0

        lax.fori_loop(0, nz, zchunk, 0)
        pltpu.sync_copy(ms_h, msv)
        m0 = msv[pl.ds(2 * c, 16)][0]
        d0 = msv[pl.ds(4 + 2 * c, 16)][0]
        m1 = msv[pl.ds(2 * c + 1, 16)][0]
        d1 = msv[pl.ds(4 + 2 * c + 1, 16)][0]
        plsc.subcore_barrier()
        rowbase = c * npad
        n_my = (n_total - s + NS - 1) // NS
        n_grp = (n_total + NS - 1) // NS
        n_grp = (n_grp + nb4 - 1) // nb4

        def group(g, _):
            ks = [g * nb4 + b for b in range(nb4)]
            safe = s * ch
            e0s = [pl.multiple_of(
                jnp.where(kk < n_my, (s + kk * NS) * ch, safe), 128)
                for kk in ks]
            cS = [pltpu.async_copy(src_h.at[pl.ds(e0s[b], ch)], sis[b],
                                   sem_idx) for b in range(nb4)]
            cD = [pltpu.async_copy(dst_h.at[pl.ds(e0s[b], ch)], dis[b],
                                   sem_idx) for b in range(nb4)]
            cA = [[pltpu.async_copy(
                lt_h.at[pl.ds(pl.multiple_of(
                    (2 * c + hl) * e + e0s[b], 128), ch)],
                avs[b].at[pl.ds(hl * ch, ch)], sem_av)
                for hl in range(2)] for b in range(nb4)]
            cG = [None] * nb4
            for b in range(nb4):
                cS[b].wait()

                def addoff(j, _):
                    o = j * 16
                    sis[b][pl.ds(o, 16)] = sis[b][pl.ds(o, 16)] + rowbase
                    return 0

                lax.fori_loop(0, ch // 16, addoff, 0)
                cG[b] = pltpu.async_copy(xw_h.at[sis[b]], ros[b], sem_gat)
            for b in range(nb4):
                cA[b][0].wait()
                cA[b][1].wait()
                for hl, (m_q, d_q) in enumerate(((m0, d0), (m1, d1))):

                    def ablk(j, _):
                        o = hl * ch + j * 16
                        v = avs[b][pl.ds(o, 16)]
                        avs[b][pl.ds(o, 16)] = jnp.exp(v - m_q) / d_q
                        return 0

                    lax.fori_loop(0, ch // 16, ablk, 0)

                @pl.when(ks[b] >= n_my)
                def _(b=b):
                    zv16 = jnp.zeros((16,), F32)

                    def zab(j, _):
                        avs[b][pl.ds(j * 16, 16)] = zv16
                        return 0

                    lax.fori_loop(0, (2 * ch) // 16, zab, 0)
            cW = [None] * nb4
            for b in range(nb4):
                cG[b].wait()

                def scale(j, _):
                    o = j * 16
                    a16_0 = avs[b][pl.ds(o, 16)]
                    a16_1 = avs[b][pl.ds(ch + o, 16)]
                    for t in range(16):
                        r = o + t
                        ros[b][r, pl.ds(0, 16)] = (
                            ros[b][r, pl.ds(0, 16)] * a16_0[t])
                        ros[b][r, pl.ds(16, 16)] = (
                            ros[b][r, pl.ds(16, 16)] * a16_1[t])
                    return 0

                lax.fori_loop(0, ch // 16, scale, 0)
                cD[b].wait()
                cW[b] = pltpu.async_copy(ros[b], acc_sh.at[dis[b]],
                                         sem_sca, add=True)
            for b in range(nb4):
                cW[b].wait()
            return 0

        lax.fori_loop(0, n_grp, group, 0)
        plsc.subcore_barrier()
        pltpu.sync_copy(acc_sh.at[pl.ds(s * rows_sub, rows_sub)],
                        out_h.at[c, pl.ds(s * rows_sub, rows_sub)])

    return k(xw2flat, src, dst, lt, ms32)


def _final_mean(x0, os1, os2, n):
    B = 512
    nb = pl.cdiv(n, B)

    def body(x0_ref, o1_ref, o2_ref, out_ref):
        def cat_elu(r):
            v = jnp.concatenate([r[0], r[1]], axis=-1)
            return jnp.where(v > 0, v, jnp.exp(v) - 1.0)

        out_ref[...] = (x0_ref[...] + cat_elu(o1_ref[...])
                        + cat_elu(o2_ref[...])) * (1.0 / 3.0)

    return pl.pallas_call(
        body,
        grid=(nb,),
        in_specs=[
            pl.BlockSpec((B, 64), lambda i: (i, 0)),
            pl.BlockSpec((2, B, 32), lambda i: (0, i, 0)),
            pl.BlockSpec((2, B, 32), lambda i: (0, i, 0)),
        ],
        out_specs=pl.BlockSpec((B, 64), lambda i: (i, 0)),
        out_shape=jax.ShapeDtypeStruct((n, 64), F32),
    )(x0, os1, os2)


def _acat_split(a):
    ad = a[0, :, :HD]
    asr = a[0, :, HD:]
    eye = jnp.eye(HEADS, dtype=F32)
    blk_d = (eye[:, None, :] * ad[:, :, None]).reshape(64, HEADS)
    blk_s = (eye[:, None, :] * asr[:, :, None]).reshape(64, HEADS)
    acat = jnp.concatenate(
        [blk_d, blk_s, jnp.zeros((64, 8), F32)], axis=1)
    return acat.reshape(2, 32, 16)


def kernel(edge_index, user_table, item_table, cat_table, W1, a1, W2, a2):
    src = edge_index[0]
    dst = edge_index[1]
    e = src.shape[0]
    n = user_table.shape[0] + item_table.shape[0] + cat_table.shape[0]
    npad = ((n + 127) // 128) * 128
    n_users = user_table.shape[0]
    n_items = item_table.shape[0]

    x0 = jnp.concatenate([user_table, item_table, cat_table], axis=0)

    def layer(x_in, W, a, from_split):
        w_split = jnp.stack([W[:, 0:32], W[:, 32:64]])
        xw2, sdss = _prep_layer(x_in, w_split, _acat_split(a), npad,
                                from_split)
        lt = _edge_logits(src, dst, sdss.reshape(npad * 16), e, npad)
        ms8 = _softmax_stats(lt.reshape(HEADS, e // 128, 128), e)
        ms32 = jnp.concatenate([ms8[:, 0], jnp.zeros((24,), F32)])
        return _message_pass(xw2.reshape(2 * npad, 32), src, dst, lt, ms32,
                             e, npad)

    os1 = layer(x0, W1, a1, False)
    os2 = layer(os1, W2, a2, True)
    final = _final_mean(x0, os1, os2, n)
    return (final[:n_users], final[n_users:n_users + n_items])
